# speedup vs baseline: 2.9646x; 2.9646x over previous
import functools

import jax
import jax.numpy as jnp
from jax import lax
from jax.experimental import pallas as pl
from jax.experimental.pallas import tpu as pltpu
from jax.experimental.pallas import tpu_sc as plsc

VOCAB = 100000
DIM = 128
B = 4096
L = 50

NC = 2
NS = 16
NW = NC * NS
N = B * L
PER_W = N // NW
CHUNK = 128
NCHUNK = PER_W // CHUNK

_mesh = plsc.VectorSubcoreMesh(core_axis_name="c", subcore_axis_name="s")


@functools.partial(
    pl.kernel,
    mesh=_mesh,
    out_type=jax.ShapeDtypeStruct((N, DIM), jnp.float32),
    scratch_types=[
        pltpu.VMEM((NCHUNK, CHUNK), jnp.int32),
        pltpu.VMEM((CHUNK, DIM), jnp.float32),
        pltpu.SemaphoreType.DMA,
    ],
)
def _gather_kernel(idx_hbm, table_hbm, out_hbm, idx_v, rows_v, sem):
    wid = lax.axis_index("s") * NC + lax.axis_index("c")
    pltpu.sync_copy(idx_hbm.at[wid], idx_v)
    base = wid * PER_W

    def body(j, carry):
        pltpu.async_copy(table_hbm.at[idx_v.at[j]], rows_v, sem).wait()
        pltpu.sync_copy(rows_v, out_hbm.at[pl.ds(base + j * CHUNK, CHUNK)])
        return carry

    lax.fori_loop(0, NCHUNK, body, 0)


def kernel(indices, table):
    idx = indices.reshape(NW, NCHUNK, CHUNK)
    out = _gather_kernel(idx, table)
    return out.reshape(B, L, DIM)


# speedup vs baseline: 3.3485x; 1.1295x over previous
import functools

import jax
import jax.numpy as jnp
from jax import lax
from jax.experimental import pallas as pl
from jax.experimental.pallas import tpu as pltpu
from jax.experimental.pallas import tpu_sc as plsc

VOCAB = 100000
DIM = 128
B = 4096
L = 50

NC = 2
NS = 16
NW = NC * NS
N = B * L
PER_W = N // NW
CHUNK = 128
NCHUNK = PER_W // CHUNK
NBUF = 5
NGROUP = NCHUNK // NBUF

_mesh = plsc.VectorSubcoreMesh(core_axis_name="c", subcore_axis_name="s")


@functools.partial(
    pl.kernel,
    mesh=_mesh,
    out_type=jax.ShapeDtypeStruct((N, DIM), jnp.float32),
    scratch_types=[
        pltpu.VMEM((NCHUNK, CHUNK), jnp.int32),
        pltpu.VMEM((NBUF, CHUNK, DIM), jnp.float32),
    ]
    + [pltpu.SemaphoreType.DMA] * NBUF,
)
def _gather_kernel(idx_hbm, table_hbm, out_hbm, idx_v, rows_v, *sems):
    wid = lax.axis_index("s") * NC + lax.axis_index("c")
    pltpu.sync_copy(idx_hbm.at[wid], idx_v)
    base = wid * PER_W

    for b in range(NBUF):
        pltpu.async_copy(table_hbm.at[idx_v.at[b]], rows_v.at[b], sems[b])

    def body(g, carry):
        j0 = g * NBUF
        for b in range(NBUF):
            j = j0 + b
            pltpu.make_async_copy(
                table_hbm.at[idx_v.at[j]], rows_v.at[b], sems[b]
            ).wait()
            pltpu.sync_copy(rows_v.at[b], out_hbm.at[pl.ds(base + j * CHUNK, CHUNK)])

            @pl.when(j + NBUF < NCHUNK)
            def _():
                pltpu.async_copy(
                    table_hbm.at[idx_v.at[j + NBUF]], rows_v.at[b], sems[b]
                )

        return carry

    lax.fori_loop(0, NGROUP, body, 0)


def kernel(indices, table):
    idx = indices.reshape(NW, NCHUNK, CHUNK)
    out = _gather_kernel(idx, table)
    return out.reshape(B, L, DIM)


# speedup vs baseline: 5.9525x; 1.7776x over previous
import functools

import jax
import jax.numpy as jnp
from jax import lax
from jax.experimental import pallas as pl
from jax.experimental.pallas import tpu as pltpu
from jax.experimental.pallas import tpu_sc as plsc

VOCAB = 100000
DIM = 128
B = 4096
L = 50

NC = 2
NS = 16
NW = NC * NS
BATCH_PER_W = B // NW
CHUNK_B = 2
CHUNK = CHUNK_B * L
NCHUNK = BATCH_PER_W // CHUNK_B
NBUF = 4
NGROUP = NCHUNK // NBUF

_mesh = plsc.VectorSubcoreMesh(core_axis_name="c", subcore_axis_name="s")


@functools.partial(
    pl.kernel,
    mesh=_mesh,
    out_type=jax.ShapeDtypeStruct((B, L, DIM), jnp.float32),
    scratch_types=[
        pltpu.VMEM((NCHUNK, 128), jnp.int32),
        pltpu.VMEM((NBUF, CHUNK, DIM), jnp.float32),
    ]
    + [pltpu.SemaphoreType.DMA] * NBUF,
    compiler_params=pltpu.CompilerParams(use_tc_tiling_on_sc=True),
)
def _gather_kernel(idx_hbm, table_hbm, out_hbm, idx_v, rows_v, *sems):
    wid = lax.axis_index("s") * NC + lax.axis_index("c")
    pltpu.sync_copy(idx_hbm.at[wid], idx_v)
    b0 = wid * BATCH_PER_W

    def start_gather(j, b):
        pltpu.async_copy(
            table_hbm.at[idx_v.at[j, pl.ds(0, CHUNK)]], rows_v.at[b], sems[b]
        )

    for b in range(NBUF):
        start_gather(b, b)

    def body(g, carry):
        j0 = g * NBUF
        for b in range(NBUF):
            j = j0 + b
            pltpu.make_async_copy(
                table_hbm.at[idx_v.at[j, pl.ds(0, CHUNK)]], rows_v.at[b], sems[b]
            ).wait()
            pltpu.sync_copy(rows_v.at[b, pl.ds(0, L)], out_hbm.at[b0 + 2 * j])
            pltpu.sync_copy(rows_v.at[b, pl.ds(L, L)], out_hbm.at[b0 + 2 * j + 1])

            @pl.when(j + NBUF < NCHUNK)
            def _():
                start_gather(j + NBUF, b)

        return carry

    lax.fori_loop(0, NGROUP, body, 0)


def kernel(indices, table):
    idx = indices.reshape(B // CHUNK_B, CHUNK)
    idx = jnp.pad(idx, ((0, 0), (0, 128 - CHUNK)))
    idx = idx.reshape(NW, NCHUNK, 128)
    return _gather_kernel(idx, table)


# speedup vs baseline: 10.5226x; 1.7678x over previous
import functools

import jax
import jax.numpy as jnp
from jax import lax
from jax.experimental import pallas as pl
from jax.experimental.pallas import tpu as pltpu
from jax.experimental.pallas import tpu_sc as plsc

VOCAB = 100000
DIM = 128
B = 4096
L = 50

NC = 2
NS = 16
NW = NC * NS
N = B * L
PER_W = N // NW
CHUNK = 128
NCHUNK = PER_W // CHUNK
NBUF = 5
NGROUP = NCHUNK // NBUF

_mesh = plsc.VectorSubcoreMesh(core_axis_name="c", subcore_axis_name="s")


@functools.partial(
    pl.kernel,
    mesh=_mesh,
    out_type=jax.ShapeDtypeStruct((L, B, DIM), jnp.float32),
    scratch_types=[
        pltpu.VMEM((NCHUNK, CHUNK), jnp.int32),
        pltpu.VMEM((NBUF, CHUNK, DIM), jnp.float32),
    ]
    + [pltpu.SemaphoreType.DMA] * NBUF,
    compiler_params=pltpu.CompilerParams(use_tc_tiling_on_sc=True),
)
def _gather_kernel(idx_hbm, table_hbm, out_hbm, idx_v, rows_v, *sems):
    wid = lax.axis_index("s") * NC + lax.axis_index("c")
    pltpu.sync_copy(idx_hbm.at[wid], idx_v)
    base = wid * PER_W

    def start_gather(j, b):
        pltpu.async_copy(table_hbm.at[idx_v.at[j]], rows_v.at[b], sems[b])

    for b in range(NBUF):
        start_gather(b, b)

    def body(g, carry):
        j0 = g * NBUF
        for b in range(NBUF):
            j = j0 + b
            pltpu.make_async_copy(
                table_hbm.at[idx_v.at[j]], rows_v.at[b], sems[b]
            ).wait()
            r = base + j * CHUNK
            pltpu.sync_copy(
                rows_v.at[b], out_hbm.at[r // B, pl.ds(lax.rem(r, B), CHUNK)]
            )

            @pl.when(j + NBUF < NCHUNK)
            def _():
                start_gather(j + NBUF, b)

        return carry

    lax.fori_loop(0, NGROUP, body, 0)


def kernel(indices, table):
    idx = indices.T.reshape(NW, NCHUNK, CHUNK)
    out = _gather_kernel(idx, table)
    return out.transpose(1, 0, 2)


# speedup vs baseline: 10.5479x; 1.0024x over previous
import functools

import jax
import jax.numpy as jnp
from jax import lax
from jax.experimental import pallas as pl
from jax.experimental.pallas import tpu as pltpu
from jax.experimental.pallas import tpu_sc as plsc

VOCAB = 100000
DIM = 128
B = 4096
L = 50

NC = 2
NS = 16
NW = NC * NS
N = B * L
PER_W = N // NW
CHUNK = 128
NCHUNK = PER_W // CHUNK
NBUF = 5
G = 2
NGROUP = NCHUNK // NBUF

_mesh = plsc.VectorSubcoreMesh(core_axis_name="c", subcore_axis_name="s")


@functools.partial(
    pl.kernel,
    mesh=_mesh,
    out_type=jax.ShapeDtypeStruct((L, B, DIM), jnp.float32),
    scratch_types=[
        pltpu.VMEM((NCHUNK, CHUNK), jnp.int32),
        pltpu.VMEM((NBUF, CHUNK, DIM), jnp.float32),
    ]
    + [pltpu.SemaphoreType.DMA] * (2 * NBUF),
    compiler_params=pltpu.CompilerParams(use_tc_tiling_on_sc=True),
)
def _gather_kernel(idx_hbm, table_hbm, out_hbm, idx_v, rows_v, *sems):
    gsems, ssems = sems[:NBUF], sems[NBUF:]
    wid = lax.axis_index("s") * NC + lax.axis_index("c")
    pltpu.sync_copy(idx_hbm.at[wid], idx_v)
    base = wid * PER_W

    def start_gather(j, b):
        pltpu.async_copy(table_hbm.at[idx_v.at[j]], rows_v.at[b], gsems[b])

    def out_slice(j):
        r = base + j * CHUNK
        return out_hbm.at[r // B, pl.ds(lax.rem(r, B), CHUNK)]

    for b in range(G):
        start_gather(b, b)

    def body(g, carry):
        j0 = g * NBUF
        for b in range(NBUF):
            j = j0 + b
            pltpu.make_async_copy(
                table_hbm.at[idx_v.at[j]], rows_v.at[b], gsems[b]
            ).wait()
            pltpu.async_copy(rows_v.at[b], out_slice(j), ssems[b])

            bn = (b + G) % NBUF

            @pl.when(j + G < NCHUNK)
            def _():
                @pl.when(j + G - NBUF >= 0)
                def _():
                    pltpu.make_async_copy(
                        rows_v.at[bn], out_slice(j + G - NBUF), ssems[bn]
                    ).wait()

                start_gather(j + G, bn)

        return carry

    lax.fori_loop(0, NGROUP, body, 0)

    for b in range(NBUF):
        j = NCHUNK - NBUF + b
        pltpu.make_async_copy(rows_v.at[b], out_slice(j), ssems[b]).wait()


def kernel(indices, table):
    idx = indices.T.reshape(NW, NCHUNK, CHUNK)
    out = _gather_kernel(idx, table)
    return out.transpose(1, 0, 2)
